# Initial kernel scaffold; baseline (speedup 1.0000x reference)
import jax
import jax.numpy as jnp
from jax.experimental import pallas as pl


def kernel(features, coords):
    raise NotImplementedError("write your pallas kernel here")


# speedup vs baseline: 112.6107x; 112.6107x over previous
import functools

import jax
import jax.numpy as jnp
from jax import lax
from jax.experimental import pallas as pl
from jax.experimental.pallas import tpu as pltpu
from jax.experimental.pallas import tpu_sc as plsc

S = 128
N_VOX = 100000
HALF = 64 * S * S
DUMP = 1024
SPM = HALF + DUMP
NSUB = 16
NCORE = 2
CHUNK = 6400
NPAD = NSUB * CHUNK
ROWS = CHUNK // 128
TILE_SHARE = SPM // NSUB
OUT_SHARE = HALF // NSUB
BOUNCE = 16384


def _scatter_body(x0_hbm, x1_hbm, x2_hbm, f_hbm, out_hbm,
                  xv, yv, zv, fv, idxv, bounce, shared):
    c = lax.axis_index("c")
    s = lax.axis_index("s")
    base = s * CHUNK

    def zstep(i, carry):
        bounce[pl.ds(i * 16, 16)] = jnp.zeros((16,), jnp.float32)
        return carry

    lax.fori_loop(0, BOUNCE // 16, zstep, 0)
    zb = s * TILE_SHARE
    for t in range(TILE_SHARE // BOUNCE):
        pltpu.sync_copy(bounce, shared.at[pl.ds(zb + t * BOUNCE, BOUNCE)])
    _rem = TILE_SHARE % BOUNCE
    if _rem:
        pltpu.sync_copy(bounce.at[pl.ds(0, _rem)],
                        shared.at[pl.ds(zb + TILE_SHARE - _rem, _rem)])

    pltpu.sync_copy(x0_hbm.at[pl.ds(base, CHUNK)], xv)
    pltpu.sync_copy(x1_hbm.at[pl.ds(base, CHUNK)], yv)
    pltpu.sync_copy(x2_hbm.at[pl.ds(base, CHUNK)], zv)
    pltpu.sync_copy(f_hbm.at[pl.ds(base, CHUNK)], fv)

    off = c * HALF

    def row(j, carry):
        for k in range(8):
            o = k * 16
            a = xv[pl.ds(j * 128 + o, 16)]
            b = yv[pl.ds(j * 128 + o, 16)]
            d = zv[pl.ds(j * 128 + o, 16)]
            flat = a * (S * S) + b * S + d
            loc = flat - off
            ok = (loc >= 0) & (loc < HALF)
            idxv[j, pl.ds(o, 16)] = jnp.where(ok, loc, HALF)
        return carry

    lax.fori_loop(0, ROWS, row, 0)

    plsc.subcore_barrier()

    for j in range(ROWS):
        pltpu.sync_copy(fv.at[pl.ds(j * 128, 128)],
                        shared.at[idxv.at[j]], add=True)

    plsc.subcore_barrier()

    for t in range(OUT_SHARE // BOUNCE):
        pltpu.sync_copy(shared.at[pl.ds(s * OUT_SHARE + t * BOUNCE, BOUNCE)],
                        bounce)
        pltpu.sync_copy(
            bounce,
            out_hbm.at[pl.ds(c * HALF + s * OUT_SHARE + t * BOUNCE, BOUNCE)])


@functools.cache
def _scatter_fn():
    return pl.kernel(
        _scatter_body,
        out_type=jax.ShapeDtypeStruct((NCORE * HALF,), jnp.float32),
        mesh=plsc.VectorSubcoreMesh(core_axis_name="c", subcore_axis_name="s",
                                    num_cores=NCORE, num_subcores=NSUB),
        scratch_types=[
            pltpu.VMEM((CHUNK,), jnp.int32),
            pltpu.VMEM((CHUNK,), jnp.int32),
            pltpu.VMEM((CHUNK,), jnp.int32),
            pltpu.VMEM((CHUNK,), jnp.float32),
            pltpu.VMEM((ROWS, 128), jnp.int32),
            pltpu.VMEM((BOUNCE,), jnp.float32),
            pltpu.VMEM_SHARED((SPM,), jnp.float32),
        ],
    )


def _box5(x, axis):
    pads = [(0, 0)] * 3
    pads[axis] = (2, 2)
    p = jnp.pad(x, pads)
    sl = [slice(None)] * 3
    acc = None
    for d in range(5):
        sl[axis] = slice(d, d + x.shape[axis])
        t = p[tuple(sl)]
        acc = t if acc is None else acc + t
    return acc


def _conv_z_body(x_ref, o_ref):
    o_ref[...] = _box5(x_ref[...], 0)


def _conv_yx_body(x_ref, o_ref):
    o_ref[...] = _box5(_box5(x_ref[...], 1), 2)


def _conv(dense3d):
    t = pl.pallas_call(
        _conv_z_body,
        grid=(8,),
        in_specs=[pl.BlockSpec((S, S // 8, S), lambda j: (0, j, 0))],
        out_specs=pl.BlockSpec((S, S // 8, S), lambda j: (0, j, 0)),
        out_shape=jax.ShapeDtypeStruct((S, S, S), jnp.float32),
    )(dense3d)
    return pl.pallas_call(
        _conv_yx_body,
        grid=(8,),
        in_specs=[pl.BlockSpec((S // 8, S, S), lambda i: (i, 0, 0))],
        out_specs=pl.BlockSpec((S // 8, S, S), lambda i: (i, 0, 0)),
        out_shape=jax.ShapeDtypeStruct((S, S, S), jnp.float32),
    )(t)


def kernel(features, coords):
    pad = NPAD - N_VOX
    f = jnp.concatenate([features[:, 0], jnp.zeros((pad,), jnp.float32)])
    cz = jnp.zeros((pad,), jnp.int32)
    x0 = jnp.concatenate([coords[:, 0], cz])
    x1 = jnp.concatenate([coords[:, 1], cz])
    x2 = jnp.concatenate([coords[:, 2], cz])
    dense = _scatter_fn()(x0, x1, x2, f)
    out = _conv(dense.reshape(S, S, S))
    return out[None, :, :, :, None]


# speedup vs baseline: 231.4788x; 2.0556x over previous
import functools

import jax
import jax.numpy as jnp
from jax import lax
from jax.experimental import pallas as pl
from jax.experimental.pallas import tpu as pltpu
from jax.experimental.pallas import tpu_sc as plsc

S = 128
N_VOX = 100000
HALF = 64 * S * S
DUMP = 1024
SPM = HALF + DUMP
NSUB = 16
NCORE = 2
CHUNK = 6272
NPAD = NSUB * CHUNK
ROWS = CHUNK // 128
TILE_SHARE = SPM // NSUB
OUT_SHARE = HALF // NSUB
BOUNCE = 16384


def _scatter_body(x0_hbm, x1_hbm, x2_hbm, f_hbm, out_hbm,
                  xv, yv, zv, fv, idxv, b0, b1, shared,
                  sem_s, sem_z, sem_a, sem_i, sem_o):
    c = lax.axis_index("c")
    s = lax.axis_index("s")
    base = s * CHUNK

    st0 = pltpu.async_copy(x0_hbm.at[pl.ds(base, CHUNK)], xv, sem_s)
    st1 = pltpu.async_copy(x1_hbm.at[pl.ds(base, CHUNK)], yv, sem_s)
    st2 = pltpu.async_copy(x2_hbm.at[pl.ds(base, CHUNK)], zv, sem_s)
    st3 = pltpu.async_copy(f_hbm.at[pl.ds(base, CHUNK)], fv, sem_s)

    def zstep(i, carry):
        b0[pl.ds(i * 16, 16)] = jnp.zeros((16,), jnp.float32)
        return carry

    lax.fori_loop(0, BOUNCE // 16, zstep, 0)
    zb = s * TILE_SHARE
    zcps = []
    for t in range(TILE_SHARE // BOUNCE):
        zcps.append(pltpu.async_copy(
            b0, shared.at[pl.ds(zb + t * BOUNCE, BOUNCE)], sem_z))
    _rem = TILE_SHARE % BOUNCE
    if _rem:
        zcps.append(pltpu.async_copy(
            b0.at[pl.ds(0, _rem)],
            shared.at[pl.ds(zb + TILE_SHARE - _rem, _rem)], sem_z))

    st0.wait(); st1.wait(); st2.wait(); st3.wait()
    off = c * HALF
    lane = lax.iota(jnp.int32, 16)

    def row(j, carry):
        for k in range(8):
            o = k * 16
            a = xv[pl.ds(j * 128 + o, 16)]
            b = yv[pl.ds(j * 128 + o, 16)]
            d = zv[pl.ds(j * 128 + o, 16)]
            flat = a * (S * S) + b * S + d
            loc = flat - off
            ok = (loc >= 0) & (loc < HALF)
            dump = HALF + ((j * 128 + o + lane) & (DUMP - 1))
            idxv[j, pl.ds(o, 16)] = jnp.where(ok, loc, dump)
        return carry

    lax.fori_loop(0, ROWS, row, 0)

    for cp in zcps:
        cp.wait()

    plsc.subcore_barrier()

    acps = [pltpu.async_copy(fv.at[pl.ds(j * 128, 128)],
                             shared.at[idxv.at[j]], sem_a, add=True)
            for j in range(ROWS)]
    for cp in acps:
        cp.wait()

    plsc.subcore_barrier()

    nchunks = OUT_SHARE // BOUNCE
    bufs = [b0, b1]
    src = s * OUT_SHARE
    dst = c * HALF + s * OUT_SHARE
    incs = [pltpu.async_copy(shared.at[pl.ds(src, BOUNCE)], b0, sem_i)]
    outs = []
    for t in range(nchunks):
        incs[t].wait()
        if t + 1 < nchunks:
            if t >= 1:
                outs[t - 1].wait()
            incs.append(pltpu.async_copy(
                shared.at[pl.ds(src + (t + 1) * BOUNCE, BOUNCE)],
                bufs[(t + 1) % 2], sem_i))
        outs.append(pltpu.async_copy(
            bufs[t % 2], out_hbm.at[pl.ds(dst + t * BOUNCE, BOUNCE)], sem_o))
    outs[nchunks - 2].wait()
    outs[nchunks - 1].wait()


@functools.cache
def _scatter_fn():
    return pl.kernel(
        _scatter_body,
        out_type=jax.ShapeDtypeStruct((NCORE * HALF,), jnp.float32),
        mesh=plsc.VectorSubcoreMesh(core_axis_name="c", subcore_axis_name="s",
                                    num_cores=NCORE, num_subcores=NSUB),
        scratch_types=[
            pltpu.VMEM((CHUNK,), jnp.int32),
            pltpu.VMEM((CHUNK,), jnp.int32),
            pltpu.VMEM((CHUNK,), jnp.int32),
            pltpu.VMEM((CHUNK,), jnp.float32),
            pltpu.VMEM((ROWS, 128), jnp.int32),
            pltpu.VMEM((BOUNCE,), jnp.float32),
            pltpu.VMEM((BOUNCE,), jnp.float32),
            pltpu.VMEM_SHARED((SPM,), jnp.float32),
            pltpu.SemaphoreType.DMA,
            pltpu.SemaphoreType.DMA,
            pltpu.SemaphoreType.DMA,
            pltpu.SemaphoreType.DMA,
            pltpu.SemaphoreType.DMA,
        ],
    )


def _box5(x, axis):
    pads = [(0, 0)] * 3
    pads[axis] = (2, 2)
    p = jnp.pad(x, pads)
    sl = [slice(None)] * 3
    acc = None
    for d in range(5):
        sl[axis] = slice(d, d + x.shape[axis])
        t = p[tuple(sl)]
        acc = t if acc is None else acc + t
    return acc


def _conv_body(top_ref, x_ref, bot_ref, o_ref):
    i = pl.program_id(0)
    top = jnp.where(i == 0, 0.0, top_ref[...])
    bot = jnp.where(i == 7, 0.0, bot_ref[...])
    ext = jnp.concatenate([top, x_ref[...], bot], axis=0)
    nz = x_ref.shape[0]
    z = (ext[0:nz] + ext[1:nz + 1] + ext[2:nz + 2]
         + ext[3:nz + 3] + ext[4:nz + 4])
    o_ref[...] = _box5(_box5(z, 1), 2)


def _conv(dense3d):
    bz = S // 8
    return pl.pallas_call(
        _conv_body,
        grid=(8,),
        in_specs=[
            pl.BlockSpec((2, S, S), lambda i: (lax.max(8 * i - 1, 0), 0, 0)),
            pl.BlockSpec((bz, S, S), lambda i: (i, 0, 0)),
            pl.BlockSpec((2, S, S),
                         lambda i: (lax.min(8 * i + 8, S // 2 - 1), 0, 0)),
        ],
        out_specs=pl.BlockSpec((bz, S, S), lambda i: (i, 0, 0)),
        out_shape=jax.ShapeDtypeStruct((S, S, S), jnp.float32),
    )(dense3d, dense3d, dense3d)


def kernel(features, coords):
    pad = NPAD - N_VOX
    f = jnp.concatenate([features[:, 0], jnp.zeros((pad,), jnp.float32)])
    cz = jnp.zeros((pad,), jnp.int32)
    x0 = jnp.concatenate([coords[:, 0], cz])
    x1 = jnp.concatenate([coords[:, 1], cz])
    x2 = jnp.concatenate([coords[:, 2], cz])
    dense = _scatter_fn()(x0, x1, x2, f)
    out = _conv(dense.reshape(S, S, S))
    return out[None, :, :, :, None]
